# baseline (speedup 1.0000x reference)
import functools

import jax
import jax.numpy as jnp
from jax import lax
from jax.experimental import pallas as pl
from jax.experimental.pallas import tpu as pltpu
from jax.experimental.pallas import tpu_sc as plsc

_EMB = 32
_HID = 128
_B = 16384
_NG, _NA, _NO = 2, 7, 21
_NCOMBO = _NG * _NA * _NO
_NPAD = 304
_BR = 128

_NC, _NS, _L = 2, 16, 16
_NW = _NC * _NS
_BPW = _B // _NW
_CHUNK = 128
_NCH = _BPW // _CHUNK


def _table_body(g_ref, a_ref, o_ref, gt_ref, at_ref, ot_ref,
                w1_ref, b1_ref, w2_ref, b2_ref, out_ref, idx_ref):
    idx_ref[...] = (g_ref[...] * (_NA * _NO) + a_ref[...] * _NO + o_ref[...])

    r = lax.broadcasted_iota(jnp.int32, (_NPAD, 1), 0)
    g = r // (_NA * _NO)
    a = (r // _NO) % _NA
    o = r % _NO
    og = (g == lax.broadcasted_iota(jnp.int32, (_NPAD, _NG), 1)).astype(jnp.float32)
    oa = (a == lax.broadcasted_iota(jnp.int32, (_NPAD, _NA), 1)).astype(jnp.float32)
    oo = (o == lax.broadcasted_iota(jnp.int32, (_NPAD, _NO), 1)).astype(jnp.float32)
    xg = jnp.dot(og, gt_ref[...], preferred_element_type=jnp.float32)
    xa = jnp.dot(oa, at_ref[...], preferred_element_type=jnp.float32)
    xo = jnp.dot(oo, ot_ref[...], preferred_element_type=jnp.float32)
    x = jnp.concatenate([xg, xa, xo], axis=1)
    h = jnp.dot(x, w1_ref[...], preferred_element_type=jnp.float32) + b1_ref[...]
    h = jnp.maximum(h, 0.0)
    out = jnp.dot(h, w2_ref[...], preferred_element_type=jnp.float32) + b2_ref[...]
    norm = jnp.sqrt(jnp.sum(out * out, axis=1, keepdims=True))
    out = out / jnp.maximum(norm, 1e-12)
    out_ref[...] = jnp.concatenate(
        [out, jnp.zeros((_NPAD, _HID - _EMB), jnp.float32)], axis=1)


_combo_table = pl.pallas_call(
    _table_body,
    out_shape=(
        jax.ShapeDtypeStruct((_NPAD, _HID), jnp.float32),
        jax.ShapeDtypeStruct((_BR, _BR), jnp.int32),
    ),
)


def _sc_gather_body(idx_hbm, tbl_hbm, out_hbm, idx_v, rows_v, gsem, wsem):
    wid = lax.axis_index("s") * _NC + lax.axis_index("c")
    base = wid * _BPW
    pltpu.sync_copy(idx_hbm.at[pl.ds(wid * _NCH, _NCH)], idx_v)
    gathers = [
        pltpu.async_copy(
            tbl_hbm.at[idx_v.at[c]],
            rows_v.at[pl.ds(c * _CHUNK, _CHUNK)],
            gsem,
        )
        for c in range(_NCH)
    ]
    writes = []
    for c in range(_NCH):
        gathers[c].wait()
        writes.append(
            pltpu.async_copy(
                rows_v.at[pl.ds(c * _CHUNK, _CHUNK)],
                out_hbm.at[pl.ds(base + c * _CHUNK, _CHUNK)],
                wsem,
            )
        )
    for w in writes:
        w.wait()


@functools.cache
def _make_sc_gather():
    mesh = plsc.VectorSubcoreMesh(
        core_axis_name="c", subcore_axis_name="s", num_cores=_NC)
    return pl.kernel(
        _sc_gather_body,
        out_type=jax.ShapeDtypeStruct((_B, _HID), jnp.float32),
        mesh=mesh,
        compiler_params=pltpu.CompilerParams(use_tc_tiling_on_sc=False),
        scratch_types=[
            pltpu.VMEM((_NCH, _CHUNK), jnp.int32),
            pltpu.VMEM((_BPW, _HID), jnp.float32),
            pltpu.SemaphoreType.DMA,
            pltpu.SemaphoreType.DMA,
        ],
    )


def kernel(gender, age, occupation, gender_table, age_table, occ_table,
           W1, b1, W2, b2):
    tbl, idx = _combo_table(
        gender.astype(jnp.int32).reshape(_BR, _BR),
        age.astype(jnp.int32).reshape(_BR, _BR),
        occupation.astype(jnp.int32).reshape(_BR, _BR),
        gender_table, age_table, occ_table,
        W1, b1.reshape(1, _HID), W2, b2.reshape(1, _EMB),
    )
    return _make_sc_gather()(idx, tbl)[:, : _EMB]
